# Initial kernel scaffold; baseline (speedup 1.0000x reference)
import jax
import jax.numpy as jnp
from jax.experimental import pallas as pl


def kernel(interaction, user_idx, item_idx, assessmentItemID, testId, KnowledgeTag, elapsed, time_diff, user_emb, item_emb, emb_interaction, emb_assess, emb_test, emb_tag, W_comb, b_comb, W_enc, b_enc):
    raise NotImplementedError("write your pallas kernel here")


# speedup vs baseline: 3.3155x; 3.3155x over previous
import functools

import jax
import jax.numpy as jnp
from jax import lax
from jax.experimental import pallas as pl
from jax.experimental.pallas import tpu as pltpu
from jax.experimental.pallas import tpu_sc as plsc

HD = 192
ED = 64
TB = 2048
CH = 128


def _sc_gather5(tables, idxs, T):
    info = plsc.get_sparse_core_info()
    NC, NS = info.num_cores, info.num_subcores
    NW = NC * NS
    per_w = T // NW
    n_ch = per_w // CH
    mesh = plsc.VectorSubcoreMesh(core_axis_name="c", subcore_axis_name="s")

    @functools.partial(
        pl.kernel,
        mesh=mesh,
        compiler_params=pltpu.CompilerParams(use_tc_tiling_on_sc=False),
        out_type=[jax.ShapeDtypeStruct((T, ED), jnp.float32) for _ in range(5)],
        scratch_types=[
            pltpu.VMEM((CH,), jnp.int32),
            pltpu.VMEM((CH, ED), jnp.float32),
            pltpu.SemaphoreType.DMA,
        ],
    )
    def k(t0, t1, t2, t3, t4, i0, i1, i2, i3, i4,
          o0, o1, o2, o3, o4, idx_v, rows_v, sem):
        wid = lax.axis_index("s") * NC + lax.axis_index("c")
        base = wid * per_w
        for tab, idx, out in ((t0, i0, o0), (t1, i1, o1), (t2, i2, o2),
                              (t3, i3, o3), (t4, i4, o4)):
            def body(c, carry, tab=tab, idx=idx, out=out):
                off = base + c * CH
                pltpu.sync_copy(idx.at[pl.ds(off, CH)], idx_v)
                pltpu.async_copy(tab.at[idx_v], rows_v, sem).wait()
                pltpu.sync_copy(rows_v, out.at[pl.ds(off, CH)])
                return carry
            lax.fori_loop(0, n_ch, body, 0)

    return k(*tables, *idxs)


def _tc_body(inter_r, el_r, td_r, ga, gt, gg, gu, gi,
             eint, wci, wca, wct, wcg, wcel, wctd, wcu, wcit,
             wea, wet, weg, weel, wetd, bc, be, eo, xo):
    a = ga[...]
    t = gt[...]
    g = gg[...]
    f32 = jnp.float32
    x = jnp.dot(a, wca[...], preferred_element_type=f32)
    x += jnp.dot(t, wct[...], preferred_element_type=f32)
    x += jnp.dot(g, wcg[...], preferred_element_type=f32)
    x += jnp.dot(gu[...], wcu[...], preferred_element_type=f32)
    x += jnp.dot(gi[...], wcit[...], preferred_element_type=f32)
    m3 = jnp.dot(eint[...], wci[...], preferred_element_type=f32)
    ii = inter_r[0, 0, :][:, None]
    x += jnp.where(ii == 0, 1.0, 0.0) * m3[0:1, :]
    x += jnp.where(ii == 1, 1.0, 0.0) * m3[1:2, :]
    x += jnp.where(ii == 2, 1.0, 0.0) * m3[2:3, :]
    el = el_r[0, 0, :][:, None]
    td = td_r[0, 0, :][:, None]
    x += el * wcel[...]
    x += td * wctd[...]
    x += bc[...]
    xo[...] = x

    e = jnp.dot(a, wea[...], preferred_element_type=f32)
    e += jnp.dot(t, wet[...], preferred_element_type=f32)
    e += jnp.dot(g, weg[...], preferred_element_type=f32)
    e += el * weel[...]
    e += td * wetd[...]
    e += be[...]
    eo[...] = e


def _tc_project(inter3, el3, td3, ga, gt, gg, gu, gi, eint, wblocks, T):
    NB = T // TB
    row_spec = pl.BlockSpec((TB, ED), lambda i: (i, 0))
    tok_spec = pl.BlockSpec((1, 1, TB), lambda i: (i, 0, 0))
    full = lambda s: pl.BlockSpec(s, lambda i: (0, 0))
    in_specs = (
        [tok_spec, tok_spec, tok_spec]
        + [row_spec] * 5
        + [full(w.shape) for w in ([eint] + list(wblocks))]
    )
    out_specs = [pl.BlockSpec((TB, HD), lambda i: (i, 0))] * 2
    out_shape = [jax.ShapeDtypeStruct((T, HD), jnp.float32)] * 2
    return pl.pallas_call(
        _tc_body,
        grid=(NB,),
        in_specs=in_specs,
        out_specs=out_specs,
        out_shape=out_shape,
    )(inter3, el3, td3, ga, gt, gg, gu, gi, eint, *wblocks)


def kernel(interaction, user_idx, item_idx, assessmentItemID, testId, KnowledgeTag,
           elapsed, time_diff, user_emb, item_emb, emb_interaction, emb_assess,
           emb_test, emb_tag, W_comb, b_comb, W_enc, b_enc):
    B, S = interaction.shape
    T = B * S
    NB = T // TB

    i32 = jnp.int32
    idx_a = assessmentItemID.reshape(-1).astype(i32)
    idx_t = testId.reshape(-1).astype(i32)
    idx_g = KnowledgeTag.reshape(-1).astype(i32)
    idx_u = user_idx.reshape(-1).astype(i32)
    idx_i = item_idx.reshape(-1).astype(i32)

    ga, gt, gg, gu, gi = _sc_gather5(
        (emb_assess, emb_test, emb_tag, user_emb, item_emb),
        (idx_a, idx_t, idx_g, idx_u, idx_i), T)

    inter3 = interaction.reshape(NB, 1, TB).astype(i32)
    el3 = elapsed.reshape(NB, 1, TB)
    td3 = time_diff.reshape(NB, 1, TB)

    wblocks = (
        W_comb[0:64],
        W_comb[64:128],
        W_comb[128:192],
        W_comb[192:256],
        W_comb[256:257],
        W_comb[257:258],
        W_comb[258:322],
        W_comb[322:386],
        W_enc[0:64],
        W_enc[64:128],
        W_enc[128:192],
        W_enc[192:193],
        W_enc[193:194],
        b_comb.reshape(1, HD),
        b_enc.reshape(1, HD),
    )
    enc_x, x = _tc_project(inter3, el3, td3, ga, gt, gg, gu, gi,
                           emb_interaction, wblocks, T)
    return (enc_x.reshape(B, S, HD), x.reshape(B, S, HD))


# speedup vs baseline: 3.8586x; 1.1638x over previous
import functools

import jax
import jax.numpy as jnp
from jax import lax
from jax.experimental import pallas as pl
from jax.experimental.pallas import tpu as pltpu
from jax.experimental.pallas import tpu_sc as plsc

HD = 192
ED = 64
NBATCH = 1024
PP = 2
TB = PP * NBATCH
CH = 128


def _sc_gather5(tables, idxs, T):
    info = plsc.get_sparse_core_info()
    NC, NS = info.num_cores, info.num_subcores
    NW = NC * NS
    per_w = T // NW
    n_ch = per_w // CH
    mesh = plsc.VectorSubcoreMesh(core_axis_name="c", subcore_axis_name="s")

    @functools.partial(
        pl.kernel,
        mesh=mesh,
        compiler_params=pltpu.CompilerParams(use_tc_tiling_on_sc=False),
        out_type=[jax.ShapeDtypeStruct((T, ED), jnp.float32) for _ in range(5)],
        scratch_types=[
            pltpu.VMEM((CH,), jnp.int32),
            pltpu.VMEM((CH, ED), jnp.float32),
            pltpu.SemaphoreType.DMA,
        ],
    )
    def k(t0, t1, t2, t3, t4, i0, i1, i2, i3, i4,
          o0, o1, o2, o3, o4, idx_v, rows_v, sem):
        wid = lax.axis_index("s") * NC + lax.axis_index("c")
        base = wid * per_w
        for tab, idx, out in ((t0, i0, o0), (t1, i1, o1), (t2, i2, o2),
                              (t3, i3, o3), (t4, i4, o4)):
            def body(c, carry, tab=tab, idx=idx, out=out):
                off = base + c * CH
                pltpu.sync_copy(idx.at[pl.ds(off, CH)], idx_v)
                pltpu.async_copy(tab.at[idx_v], rows_v, sem).wait()
                pltpu.sync_copy(rows_v, out.at[pl.ds(off, CH)])
                return carry
            lax.fori_loop(0, n_ch, body, 0)

    return k(*tables, *idxs)


def _tc_body(inter_r, el_r, td_r, ga, gt, gg, gu, gi,
             eint, wci, wca, wct, wcg, wcel, wctd, wcu, wcit,
             wea, wet, weg, weel, wetd, bc, be, eo, xo):
    a = ga[...]
    t = gt[...]
    g = gg[...]
    f32 = jnp.float32
    x = jnp.dot(a, wca[...], preferred_element_type=f32)
    x += jnp.dot(t, wct[...], preferred_element_type=f32)
    x += jnp.dot(g, wcg[...], preferred_element_type=f32)
    x += jnp.dot(gu[...], wcu[...], preferred_element_type=f32)
    x += jnp.dot(gi[...], wcit[...], preferred_element_type=f32)
    m3 = jnp.dot(eint[...], wci[...], preferred_element_type=f32)
    ii = inter_r[0, 0, :][:, None]
    x += jnp.where(ii == 0, 1.0, 0.0) * m3[0:1, :]
    x += jnp.where(ii == 1, 1.0, 0.0) * m3[1:2, :]
    x += jnp.where(ii == 2, 1.0, 0.0) * m3[2:3, :]
    el = el_r[0, 0, :][:, None]
    td = td_r[0, 0, :][:, None]
    x += el * wcel[...]
    x += td * wctd[...]
    x += bc[...]

    e = jnp.dot(a, wea[...], preferred_element_type=f32)
    e += jnp.dot(t, wet[...], preferred_element_type=f32)
    e += jnp.dot(g, weg[...], preferred_element_type=f32)
    e += el * weel[...]
    e += td * wetd[...]
    e += be[...]

    for p in range(PP):
        xo[p] = x[p * NBATCH:(p + 1) * NBATCH, :].T
        eo[p] = e[p * NBATCH:(p + 1) * NBATCH, :].T


def _tc_project(inter3, el3, td3, ga, gt, gg, gu, gi, eint, wblocks, T, S):
    NB = T // TB
    row_spec = pl.BlockSpec((TB, ED), lambda i: (i, 0))
    tok_spec = pl.BlockSpec((1, 1, TB), lambda i: (i, 0, 0))
    full = lambda s: pl.BlockSpec(s, lambda i: (0, 0))
    in_specs = (
        [tok_spec, tok_spec, tok_spec]
        + [row_spec] * 5
        + [full(w.shape) for w in ([eint] + list(wblocks))]
    )
    out_specs = [pl.BlockSpec((PP, HD, NBATCH), lambda i: (i, 0, 0))] * 2
    out_shape = [jax.ShapeDtypeStruct((S, HD, NBATCH), jnp.float32)] * 2
    return pl.pallas_call(
        _tc_body,
        grid=(NB,),
        in_specs=in_specs,
        out_specs=out_specs,
        out_shape=out_shape,
    )(inter3, el3, td3, ga, gt, gg, gu, gi, eint, *wblocks)


def kernel(interaction, user_idx, item_idx, assessmentItemID, testId, KnowledgeTag,
           elapsed, time_diff, user_emb, item_emb, emb_interaction, emb_assess,
           emb_test, emb_tag, W_comb, b_comb, W_enc, b_enc):
    B, S = interaction.shape
    T = B * S
    NB = T // TB

    i32 = jnp.int32
    idx_a = assessmentItemID.T.reshape(-1).astype(i32)
    idx_t = testId.T.reshape(-1).astype(i32)
    idx_g = KnowledgeTag.T.reshape(-1).astype(i32)
    idx_u = user_idx.T.reshape(-1).astype(i32)
    idx_i = item_idx.T.reshape(-1).astype(i32)

    ga, gt, gg, gu, gi = _sc_gather5(
        (emb_assess, emb_test, emb_tag, user_emb, item_emb),
        (idx_a, idx_t, idx_g, idx_u, idx_i), T)

    inter3 = interaction.T.reshape(NB, 1, TB).astype(i32)
    el3 = elapsed.T.reshape(NB, 1, TB)
    td3 = time_diff.T.reshape(NB, 1, TB)

    wblocks = (
        W_comb[0:64],
        W_comb[64:128],
        W_comb[128:192],
        W_comb[192:256],
        W_comb[256:257],
        W_comb[257:258],
        W_comb[258:322],
        W_comb[322:386],
        W_enc[0:64],
        W_enc[64:128],
        W_enc[128:192],
        W_enc[192:193],
        W_enc[193:194],
        b_comb.reshape(1, HD),
        b_enc.reshape(1, HD),
    )
    enc_x, x = _tc_project(inter3, el3, td3, ga, gt, gg, gu, gi,
                           emb_interaction, wblocks, T, S)
    return (jnp.transpose(enc_x, (2, 0, 1)), jnp.transpose(x, (2, 0, 1)))
